# baseline (speedup 1.0000x reference)
import functools

import jax
import jax.numpy as jnp
from jax import lax
from jax.experimental import pallas as pl
from jax.experimental.pallas import tpu as pltpu
from jax.experimental.pallas import tpu_sc as plsc

_NC = 2
_NS = 16
_NW = _NC * _NS
_CHUNK = 120
_LANES = 16
_FRAC0 = 0.57


def _node_pad(n_nodes: int) -> int:
  return -(-(n_nodes + 8) // (_NS * 8)) * (_NS * 8)


def _make_agg(n_nodes: int, d: int, n0: int, n1: int, with_counts: bool):
  assert min(n0, n1) >= 4
  n_pad = _node_pad(n_nodes)
  rpt = n_pad // _NS

  mesh = plsc.VectorSubcoreMesh(core_axis_name="c", subcore_axis_name="s")

  out_type = [jax.ShapeDtypeStruct((_NC * n_pad, d), jnp.float32)]
  if with_counts:
    out_type.append(jax.ShapeDtypeStruct((_NC * n_pad, d), jnp.float32))
  scratch = [
      pltpu.VMEM((3, _CHUNK), jnp.int32),
      pltpu.VMEM((4, _CHUNK), jnp.int32),
      pltpu.VMEM((3, _CHUNK, d), jnp.float32),
      pltpu.VMEM_SHARED((n_pad, d), jnp.float32),
      pltpu.SemaphoreType.DMA,
      pltpu.SemaphoreType.DMA,
      pltpu.SemaphoreType.DMA,
      pltpu.SemaphoreType.DMA,
  ]

  @functools.partial(pl.kernel, mesh=mesh, out_type=out_type,
                     scratch_types=scratch)
  def agg(table_hbm, src_hbm, dst_hbm, *refs):
    if with_counts:
      (out_hbm, cnt_hbm, srcw, dstw, rows_v, acc_sh,
       gsem, ssem, isem, dsem) = refs
    else:
      (out_hbm, srcw, dstw, rows_v, acc_sh, gsem, ssem, isem, dsem) = refs
      cnt_hbm = None

    cid = lax.axis_index("c")
    sid = lax.axis_index("s")
    nc = jnp.where(cid == 0, n0, n1)
    base = jnp.where(cid == 0, sid * n0, _NS * n0 + sid * n1)
    m0 = (n0 + n1) // 2
    m1 = n0 + n1 - m0
    mc = jnp.where(cid == 0, m0, m1)
    base_c = jnp.where(cid == 0, sid * m0, _NS * m0 + sid * m1)
    r0 = pl.multiple_of(sid * rpt, 8)
    ro = pl.multiple_of(cid * n_pad + sid * rpt, 8)

    def fill_slot(slot, val):
      def _f(i, _):
        rows_v[slot, i // (d // _LANES),
               pl.ds((i % (d // _LANES)) * _LANES, _LANES)] = (
                   jnp.full((_LANES,), val, jnp.float32))
        return 0
      lax.fori_loop(0, _CHUNK * (d // _LANES), _f, 0)

    def zero_acc_slice(slot):
      done = 0
      while done < rpt:
        step = min(_CHUNK, rpt - done)
        pltpu.sync_copy(rows_v.at[slot, pl.ds(0, step)],
                        acc_sh.at[pl.ds(r0 + done, step)])
        done += step

    def drain_gather():
      pltpu.make_async_copy(table_hbm.at[pl.ds(0, _CHUNK)], rows_v.at[0],
                            gsem).wait()

    def drain_scatter():
      pltpu.make_async_copy(rows_v.at[0], acc_sh.at[pl.ds(0, _CHUNK)],
                            ssem).wait()

    def drain_src():
      pltpu.make_async_copy(src_hbm.at[0], srcw.at[0], isem).wait()

    def drain_dst():
      pltpu.make_async_copy(dst_hbm.at[0], dstw.at[0], dsem).wait()

    def issue_src(c, slot):
      pltpu.async_copy(src_hbm.at[base + c], srcw.at[slot], isem)

    def issue_dst(c, slot):
      pltpu.async_copy(dst_hbm.at[base + c], dstw.at[slot], dsem)

    def issue_dst_c(c, slot):
      pltpu.async_copy(dst_hbm.at[base_c + c], dstw.at[slot], dsem)

    def issue_gather(slot_ids, slot_rows):
      pltpu.async_copy(table_hbm.at[srcw.at[slot_ids]], rows_v.at[slot_rows],
                       gsem)

    def issue_scatter(slot_ids, slot_rows):
      pltpu.async_copy(rows_v.at[slot_rows], acc_sh.at[dstw.at[slot_ids]],
                       ssem, add=True)

    def counts_pass():
      for k in range(4):
        issue_dst_c(k, k)

      def _b(c, _):
        @pl.when(c >= 2)
        def _():
          drain_scatter()
          @pl.when(c + 2 < mc)
          def _():
            issue_dst_c(c + 2, lax.rem(c + 2, 4))
        drain_dst()
        issue_scatter(lax.rem(c, 4), 0)
        return 0
      lax.fori_loop(0, mc, _b, 0)
      drain_scatter()
      drain_scatter()

    def feature_pass():
      pltpu.sync_copy(src_hbm.at[base], srcw.at[0])
      issue_gather(0, 0)
      issue_src(1, 1)
      for k in range(4):
        issue_dst(k, k)
      drain_src()
      issue_gather(1, 1)
      issue_src(2, 2)

      def _b(c, _):
        drain_gather()
        @pl.when(c + 3 < nc)
        def _():
          issue_src(c + 3, lax.rem(c + 3, 3))
        @pl.when(c >= 1)
        def _():
          drain_scatter()
          @pl.when(c + 3 < nc)
          def _():
            issue_dst(c + 3, lax.rem(c + 3, 4))
        @pl.when(c + 2 < nc)
        def _():
          drain_src()
          issue_gather(lax.rem(c + 2, 3), lax.rem(c + 2, 3))
        drain_dst()
        issue_scatter(lax.rem(c, 4), lax.rem(c, 3))
        return 0
      lax.fori_loop(0, nc, _b, 0)
      drain_scatter()

    def copy_out(dst_hbm_ref):
      pltpu.sync_copy(acc_sh.at[pl.ds(r0, rpt)], dst_hbm_ref.at[pl.ds(ro, rpt)])

    if with_counts:
      fill_slot(0, 1.0)
      fill_slot(1, 0.0)
      zero_acc_slice(1)
      plsc.subcore_barrier()
      counts_pass()
      plsc.subcore_barrier()
      copy_out(cnt_hbm)
      zero_acc_slice(1)
    else:
      fill_slot(1, 0.0)
      zero_acc_slice(1)
    plsc.subcore_barrier()
    feature_pass()
    plsc.subcore_barrier()
    copy_out(out_hbm)

  return agg


def _dense_lin(x, w_r, b_l):
  n, d = x.shape

  def body(x_ref, wr_ref, bl_ref, o_ref):
    o_ref[...] = jnp.dot(x_ref[...], wr_ref[...],
                         preferred_element_type=jnp.float32) + bl_ref[...]

  return pl.pallas_call(
      body,
      out_shape=jax.ShapeDtypeStruct((n, d), jnp.float32),
  )(x, w_r, b_l.reshape(1, d))


def _dense_combine(sums, cnts, xr, w_l, apply_relu):
  n, d = xr.shape
  n_pad = _node_pad(n)

  def body(s_ref, c_ref, xr_ref, wl_ref, o_ref):
    s = s_ref[...]
    s = s[:n] + s[n_pad:n_pad + n]
    c = c_ref[...]
    c = c[:n, 0:1] + c[n_pad:n_pad + n, 0:1]
    mean = s * (1.0 / jnp.maximum(c, 1.0))
    acc = jnp.dot(mean, wl_ref[...], preferred_element_type=jnp.float32)
    acc = acc + xr_ref[...]
    if apply_relu:
      acc = jnp.maximum(acc, 0.0)
    o_ref[...] = acc

  return pl.pallas_call(
      body,
      out_shape=jax.ShapeDtypeStruct((n, d), jnp.float32),
  )(sums, cnts, xr, w_l)


def kernel(x, edge_index, W_l1, b_l1, W_r1, W_l2, b_l2, W_r2):
  n, d = x.shape
  e = edge_index.shape[1]
  tot = -(-e // (_CHUNK * _NS))
  n0 = max(4, round(tot * _FRAC0))
  n1 = tot - n0
  nch = _NS * (n0 + n1)
  pad = nch * _CHUNK - e

  src2 = jnp.concatenate(
      [edge_index[0].astype(jnp.int32),
       jnp.zeros((pad,), jnp.int32)]).reshape(nch, _CHUNK)
  dst2 = jnp.concatenate(
      [edge_index[1].astype(jnp.int32),
       jnp.full((pad,), n, jnp.int32)]).reshape(nch, _CHUNK)

  agg_c = _make_agg(n, d, n0, n1, True)
  agg = _make_agg(n, d, n0, n1, False)

  xr1 = _dense_lin(x, W_r1, b_l1)
  sums1, cnts = agg_c(x, src2, dst2)
  h = _dense_combine(sums1, cnts, xr1, W_l1, True)
  xr2 = _dense_lin(h, W_r2, b_l2)
  (sums2,) = agg(h, src2, dst2)
  return _dense_combine(sums2, cnts, xr2, W_l2, False)


# speedup vs baseline: 1.0113x; 1.0113x over previous
import functools

import jax
import jax.numpy as jnp
from jax import lax
from jax.experimental import pallas as pl
from jax.experimental.pallas import tpu as pltpu
from jax.experimental.pallas import tpu_sc as plsc

_NC = 2
_NS = 16
_NW = _NC * _NS
_CHUNK = 120
_LANES = 16
_FRAC0 = 0.57


def _node_pad(n_nodes: int) -> int:
  return -(-(n_nodes + 8) // (_NS * 8)) * (_NS * 8)


def _make_agg(n_nodes: int, d: int, n0: int, n1: int, with_counts: bool):
  assert min(n0, n1) >= 4
  n_pad = _node_pad(n_nodes)
  rpt = n_pad // _NS

  mesh = plsc.VectorSubcoreMesh(core_axis_name="c", subcore_axis_name="s")

  out_type = [jax.ShapeDtypeStruct((_NC * n_pad, d), jnp.float32)]
  if with_counts:
    out_type.append(jax.ShapeDtypeStruct((_NC * n_pad, d), jnp.float32))
  scratch = [
      pltpu.VMEM((3, _CHUNK), jnp.int32),
      pltpu.VMEM((4, _CHUNK), jnp.int32),
      pltpu.VMEM((3, _CHUNK, d), jnp.float32),
      pltpu.VMEM_SHARED((n_pad, d), jnp.float32),
      pltpu.SemaphoreType.DMA,
      pltpu.SemaphoreType.DMA,
      pltpu.SemaphoreType.DMA,
      pltpu.SemaphoreType.DMA,
  ]

  @functools.partial(pl.kernel, mesh=mesh, out_type=out_type,
                     scratch_types=scratch)
  def agg(table_hbm, src_hbm, dst_hbm, *refs):
    if with_counts:
      (out_hbm, cnt_hbm, srcw, dstw, rows_v, acc_sh,
       gsem, ssem, isem, dsem) = refs
    else:
      (out_hbm, srcw, dstw, rows_v, acc_sh, gsem, ssem, isem, dsem) = refs
      cnt_hbm = None

    cid = lax.axis_index("c")
    sid = lax.axis_index("s")
    nc = jnp.where(cid == 0, n0, n1)
    base = jnp.where(cid == 0, sid * n0, _NS * n0 + sid * n1)
    m0 = n0
    m1 = n1
    mc = jnp.where(cid == 0, m0, m1)
    base_c = jnp.where(cid == 0, sid * m0, _NS * m0 + sid * m1)
    r0 = pl.multiple_of(sid * rpt, 8)
    ro = pl.multiple_of(cid * n_pad + sid * rpt, 8)

    def fill_slot(slot, val):
      def _f(i, _):
        rows_v[slot, i // (d // _LANES),
               pl.ds((i % (d // _LANES)) * _LANES, _LANES)] = (
                   jnp.full((_LANES,), val, jnp.float32))
        return 0
      lax.fori_loop(0, _CHUNK * (d // _LANES), _f, 0)

    def zero_acc_slice(slot):
      done = 0
      while done < rpt:
        step = min(_CHUNK, rpt - done)
        pltpu.sync_copy(rows_v.at[slot, pl.ds(0, step)],
                        acc_sh.at[pl.ds(r0 + done, step)])
        done += step

    def drain_gather():
      pltpu.make_async_copy(table_hbm.at[pl.ds(0, _CHUNK)], rows_v.at[0],
                            gsem).wait()

    def drain_scatter():
      pltpu.make_async_copy(rows_v.at[0], acc_sh.at[pl.ds(0, _CHUNK)],
                            ssem).wait()

    def drain_src():
      pltpu.make_async_copy(src_hbm.at[0], srcw.at[0], isem).wait()

    def drain_dst():
      pltpu.make_async_copy(dst_hbm.at[0], dstw.at[0], dsem).wait()

    def issue_src(c, slot):
      pltpu.async_copy(src_hbm.at[base + c], srcw.at[slot], isem)

    def issue_dst(c, slot):
      pltpu.async_copy(dst_hbm.at[base + c], dstw.at[slot], dsem)

    def issue_dst_c(c, slot):
      pltpu.async_copy(dst_hbm.at[base_c + c], dstw.at[slot], dsem)

    def issue_gather(slot_ids, slot_rows):
      pltpu.async_copy(table_hbm.at[srcw.at[slot_ids]], rows_v.at[slot_rows],
                       gsem)

    def issue_scatter(slot_ids, slot_rows):
      pltpu.async_copy(rows_v.at[slot_rows], acc_sh.at[dstw.at[slot_ids]],
                       ssem, add=True)

    def counts_pass():
      for k in range(4):
        issue_dst_c(k, k)

      def _b(c, _):
        @pl.when(c >= 2)
        def _():
          drain_scatter()
          @pl.when(c + 2 < mc)
          def _():
            issue_dst_c(c + 2, lax.rem(c + 2, 4))
        drain_dst()
        issue_scatter(lax.rem(c, 4), 0)
        return 0
      lax.fori_loop(0, mc, _b, 0)
      drain_scatter()
      drain_scatter()

    def feature_pass():
      pltpu.sync_copy(src_hbm.at[base], srcw.at[0])
      issue_gather(0, 0)
      issue_src(1, 1)
      for k in range(4):
        issue_dst(k, k)
      drain_src()
      issue_gather(1, 1)
      issue_src(2, 2)

      def _b(c, _):
        drain_gather()
        @pl.when(c + 3 < nc)
        def _():
          issue_src(c + 3, lax.rem(c + 3, 3))
        @pl.when(c >= 1)
        def _():
          drain_scatter()
          @pl.when(c + 3 < nc)
          def _():
            issue_dst(c + 3, lax.rem(c + 3, 4))
        @pl.when(c + 2 < nc)
        def _():
          drain_src()
          issue_gather(lax.rem(c + 2, 3), lax.rem(c + 2, 3))
        drain_dst()
        issue_scatter(lax.rem(c, 4), lax.rem(c, 3))
        return 0
      lax.fori_loop(0, nc, _b, 0)
      drain_scatter()

    def copy_out(dst_hbm_ref):
      pltpu.sync_copy(acc_sh.at[pl.ds(r0, rpt)], dst_hbm_ref.at[pl.ds(ro, rpt)])

    if with_counts:
      fill_slot(0, 1.0)
      fill_slot(1, 0.0)
      zero_acc_slice(1)
      plsc.subcore_barrier()
      counts_pass()
      plsc.subcore_barrier()
      copy_out(cnt_hbm)
      zero_acc_slice(1)
    else:
      fill_slot(1, 0.0)
      zero_acc_slice(1)
    plsc.subcore_barrier()
    feature_pass()
    plsc.subcore_barrier()
    copy_out(out_hbm)

  return agg


def _dense_lin(x, w_r, b_l):
  n, d = x.shape

  def body(x_ref, wr_ref, bl_ref, o_ref):
    o_ref[...] = jnp.dot(x_ref[...], wr_ref[...],
                         preferred_element_type=jnp.float32) + bl_ref[...]

  return pl.pallas_call(
      body,
      out_shape=jax.ShapeDtypeStruct((n, d), jnp.float32),
  )(x, w_r, b_l.reshape(1, d))


def _dense_combine(sums, cnts, xr, w_l, apply_relu):
  n, d = xr.shape
  n_pad = _node_pad(n)

  def body(s_ref, c_ref, xr_ref, wl_ref, o_ref):
    s = s_ref[...]
    s = s[:n] + s[n_pad:n_pad + n]
    c = c_ref[...]
    c = c[:n, 0:1] + c[n_pad:n_pad + n, 0:1]
    mean = s * (1.0 / jnp.maximum(c, 1.0))
    acc = jnp.dot(mean, wl_ref[...], preferred_element_type=jnp.float32)
    acc = acc + xr_ref[...]
    if apply_relu:
      acc = jnp.maximum(acc, 0.0)
    o_ref[...] = acc

  return pl.pallas_call(
      body,
      out_shape=jax.ShapeDtypeStruct((n, d), jnp.float32),
  )(sums, cnts, xr, w_l)


def kernel(x, edge_index, W_l1, b_l1, W_r1, W_l2, b_l2, W_r2):
  n, d = x.shape
  e = edge_index.shape[1]
  tot = -(-e // (_CHUNK * _NS))
  n0 = max(4, round(tot * _FRAC0))
  n1 = tot - n0
  nch = _NS * (n0 + n1)
  pad = nch * _CHUNK - e

  src2 = jnp.concatenate(
      [edge_index[0].astype(jnp.int32),
       jnp.zeros((pad,), jnp.int32)]).reshape(nch, _CHUNK)
  dst2 = jnp.concatenate(
      [edge_index[1].astype(jnp.int32),
       jnp.full((pad,), n, jnp.int32)]).reshape(nch, _CHUNK)

  agg_c = _make_agg(n, d, n0, n1, True)
  agg = _make_agg(n, d, n0, n1, False)

  xr1 = _dense_lin(x, W_r1, b_l1)
  sums1, cnts = agg_c(x, src2, dst2)
  h = _dense_combine(sums1, cnts, xr1, W_l1, True)
  xr2 = _dense_lin(h, W_r2, b_l2)
  (sums2,) = agg(h, src2, dst2)
  return _dense_combine(sums2, cnts, xr2, W_l2, False)


# speedup vs baseline: 1.0226x; 1.0111x over previous
import functools

import jax
import jax.numpy as jnp
from jax import lax
from jax.experimental import pallas as pl
from jax.experimental.pallas import tpu as pltpu
from jax.experimental.pallas import tpu_sc as plsc

_NC = 2
_NS = 16
_NW = _NC * _NS
_CHUNK = 120
_LANES = 16
_FRAC0 = 0.57


def _node_pad(n_nodes: int) -> int:
  return -(-(n_nodes + 8) // (_NS * 8)) * (_NS * 8)


def _make_agg(n_nodes: int, d: int, n0: int, n1: int, with_counts: bool):
  assert min(n0, n1) >= 4
  n_pad = _node_pad(n_nodes)
  rpt = n_pad // _NS

  mesh = plsc.VectorSubcoreMesh(core_axis_name="c", subcore_axis_name="s")

  out_type = [jax.ShapeDtypeStruct((_NC * n_pad, d), jnp.float32)]
  if with_counts:
    out_type.append(jax.ShapeDtypeStruct((_NC * n_pad, d), jnp.float32))
  scratch = [
      pltpu.VMEM((3, _CHUNK), jnp.int32),
      pltpu.VMEM((4, _CHUNK), jnp.int32),
      pltpu.VMEM((3, _CHUNK, d), jnp.float32),
      pltpu.VMEM_SHARED((n_pad, d), jnp.float32),
      pltpu.SemaphoreType.DMA,
      pltpu.SemaphoreType.DMA,
      pltpu.SemaphoreType.DMA,
      pltpu.SemaphoreType.DMA,
  ]

  @functools.partial(pl.kernel, mesh=mesh, out_type=out_type,
                     scratch_types=scratch)
  def agg(table_hbm, src_hbm, dst_hbm, *refs):
    if with_counts:
      (out_hbm, cnt_hbm, srcw, dstw, rows_v, acc_sh,
       gsem, ssem, isem, dsem) = refs
    else:
      (out_hbm, srcw, dstw, rows_v, acc_sh, gsem, ssem, isem, dsem) = refs
      cnt_hbm = None

    cid = lax.axis_index("c")
    sid = lax.axis_index("s")
    nc = jnp.where(cid == 0, n0, n1)
    base = jnp.where(cid == 0, sid * n0, _NS * n0 + sid * n1)
    m0 = n0
    m1 = n1
    mc = jnp.where(cid == 0, m0, m1)
    base_c = jnp.where(cid == 0, sid * m0, _NS * m0 + sid * m1)
    r0 = pl.multiple_of(sid * rpt, 8)
    ro = pl.multiple_of(cid * n_pad + sid * rpt, 8)

    def fill_slot(slot, val):
      def _f(i, _):
        rows_v[slot, i // (d // _LANES),
               pl.ds((i % (d // _LANES)) * _LANES, _LANES)] = (
                   jnp.full((_LANES,), val, jnp.float32))
        return 0
      lax.fori_loop(0, _CHUNK * (d // _LANES), _f, 0)

    def zero_acc_slice(slot):
      done = 0
      while done < rpt:
        step = min(_CHUNK, rpt - done)
        pltpu.sync_copy(rows_v.at[slot, pl.ds(0, step)],
                        acc_sh.at[pl.ds(r0 + done, step)])
        done += step

    def drain_gather():
      pltpu.make_async_copy(table_hbm.at[pl.ds(0, _CHUNK)], rows_v.at[0],
                            gsem).wait()

    def drain_scatter():
      pltpu.make_async_copy(rows_v.at[0], acc_sh.at[pl.ds(0, _CHUNK)],
                            ssem).wait()

    def drain_src():
      pltpu.make_async_copy(src_hbm.at[0], srcw.at[0], isem).wait()

    def drain_dst():
      pltpu.make_async_copy(dst_hbm.at[0], dstw.at[0], dsem).wait()

    def issue_src(c, slot):
      pltpu.async_copy(src_hbm.at[base + c], srcw.at[slot], isem)

    def issue_dst(c, slot):
      pltpu.async_copy(dst_hbm.at[base + c], dstw.at[slot], dsem)

    def issue_dst_c(c, slot):
      pltpu.async_copy(dst_hbm.at[base_c + c], dstw.at[slot], dsem)

    def issue_gather(slot_ids, slot_rows):
      pltpu.async_copy(table_hbm.at[srcw.at[slot_ids]], rows_v.at[slot_rows],
                       gsem)

    def issue_scatter(slot_ids, slot_rows):
      pltpu.async_copy(rows_v.at[slot_rows], acc_sh.at[dstw.at[slot_ids]],
                       ssem, add=True)

    def counts_pass():
      for k in range(4):
        issue_dst_c(k, k)

      def _b(c, _):
        @pl.when(c >= 2)
        def _():
          drain_scatter()
          @pl.when(c + 2 < mc)
          def _():
            issue_dst_c(c + 2, lax.rem(c + 2, 4))
        drain_dst()
        issue_scatter(lax.rem(c, 4), 0)
        return 0
      lax.fori_loop(0, mc, _b, 0)
      drain_scatter()
      drain_scatter()

    def feature_pass():
      pltpu.sync_copy(src_hbm.at[base], srcw.at[0])
      issue_gather(0, 0)
      issue_src(1, 1)
      for k in range(4):
        issue_dst(k, k)
      drain_src()
      issue_gather(1, 1)
      issue_src(2, 2)

      def _b(c, _):
        drain_gather()
        @pl.when(c + 3 < nc)
        def _():
          issue_src(c + 3, lax.rem(c + 3, 3))
        @pl.when(c >= 1)
        def _():
          drain_scatter()
          @pl.when(c + 3 < nc)
          def _():
            issue_dst(c + 3, lax.rem(c + 3, 4))
        @pl.when(c + 2 < nc)
        def _():
          drain_src()
          issue_gather(lax.rem(c + 2, 3), lax.rem(c + 2, 3))
        drain_dst()
        issue_scatter(lax.rem(c, 4), lax.rem(c, 3))
        return 0
      lax.fori_loop(0, nc, _b, 0)
      drain_scatter()

    def copy_out(dst_hbm_ref):
      pltpu.sync_copy(acc_sh.at[pl.ds(r0, rpt)], dst_hbm_ref.at[pl.ds(ro, rpt)])

    if with_counts:
      fill_slot(0, 1.0)
      fill_slot(1, 0.0)
      zero_acc_slice(1)
      plsc.subcore_barrier()
      counts_pass()
      plsc.subcore_barrier()
      copy_out(cnt_hbm)
      zero_acc_slice(1)
    else:
      fill_slot(1, 0.0)
      zero_acc_slice(1)
    plsc.subcore_barrier()
    feature_pass()
    plsc.subcore_barrier()
    copy_out(out_hbm)

  return agg


def _dense_lin(x, w_r, b_l):
  n, d = x.shape

  def body(x_ref, wr_ref, bl_ref, o_ref):
    o_ref[...] = jnp.dot(x_ref[...], wr_ref[...],
                         preferred_element_type=jnp.float32) + bl_ref[...]

  return pl.pallas_call(
      body,
      out_shape=jax.ShapeDtypeStruct((n, d), jnp.float32),
  )(x, w_r, b_l.reshape(1, d))


def _dense_combine1(sums, cnts, xr, w_l):
  n, d = xr.shape
  n_pad = _node_pad(n)

  def body(s_ref, c_ref, xr_ref, wl_ref, o_ref, ic_ref):
    s = s_ref[...]
    s = s[:n] + s[n_pad:n_pad + n]
    c = c_ref[...]
    c = c[:n, 0:1] + c[n_pad:n_pad + n, 0:1]
    ic = 1.0 / jnp.maximum(c, 1.0)
    ic_ref[...] = ic
    mean = s * ic
    acc = jnp.dot(mean, wl_ref[...], preferred_element_type=jnp.float32)
    o_ref[...] = jnp.maximum(acc + xr_ref[...], 0.0)

  return pl.pallas_call(
      body,
      out_shape=[jax.ShapeDtypeStruct((n, d), jnp.float32),
                 jax.ShapeDtypeStruct((n, 1), jnp.float32)],
  )(sums, cnts, xr, w_l)


def _dense_combine2(sums, invc, xr, w_l):
  n, d = xr.shape
  n_pad = _node_pad(n)

  def body(s_ref, ic_ref, xr_ref, wl_ref, o_ref):
    s = s_ref[...]
    s = s[:n] + s[n_pad:n_pad + n]
    mean = s * ic_ref[...]
    acc = jnp.dot(mean, wl_ref[...], preferred_element_type=jnp.float32)
    o_ref[...] = acc + xr_ref[...]

  return pl.pallas_call(
      body,
      out_shape=jax.ShapeDtypeStruct((n, d), jnp.float32),
  )(sums, invc, xr, w_l)


def kernel(x, edge_index, W_l1, b_l1, W_r1, W_l2, b_l2, W_r2):
  n, d = x.shape
  e = edge_index.shape[1]
  tot = -(-e // (_CHUNK * _NS))
  n0 = max(4, round(tot * _FRAC0))
  n1 = tot - n0
  nch = _NS * (n0 + n1)
  pad = nch * _CHUNK - e

  src2 = jnp.concatenate(
      [edge_index[0].astype(jnp.int32),
       jnp.zeros((pad,), jnp.int32)]).reshape(nch, _CHUNK)
  dst2 = jnp.concatenate(
      [edge_index[1].astype(jnp.int32),
       jnp.full((pad,), n, jnp.int32)]).reshape(nch, _CHUNK)

  agg_c = _make_agg(n, d, n0, n1, True)
  agg = _make_agg(n, d, n0, n1, False)

  xr1 = _dense_lin(x, W_r1, b_l1)
  sums1, cnts = agg_c(x, src2, dst2)
  h, invc = _dense_combine1(sums1, cnts, xr1, W_l1)
  xr2 = _dense_lin(h, W_r2, b_l2)
  (sums2,) = agg(h, src2, dst2)
  return _dense_combine2(sums2, invc, xr2, W_l2)
